# baseline (speedup 1.0000x reference)
import functools

import jax
import jax.numpy as jnp
from jax import lax
from jax.experimental import pallas as pl
from jax.experimental.pallas import tpu as pltpu
from jax.experimental.pallas import tpu_sc as plsc

_NUM_CORES = 2
_NUM_SUBCORES = 16
_NUM_WORKERS = _NUM_CORES * _NUM_SUBCORES
_P1_CHUNK = 98304


@functools.partial(jax.jit, static_argnums=(2,))
def _gather_t(table_t, idx_t, v):
    d, _ = table_t.shape
    l_fields, b = idx_t.shape
    m_rows = l_fields * d
    mesh = plsc.VectorSubcoreMesh(core_axis_name="c", subcore_axis_name="s")

    chunks = [(t * _P1_CHUNK, _P1_CHUNK) for t in range(v // _P1_CHUNK)]
    rem_off = (v // _P1_CHUNK) * _P1_CHUNK
    rem_aligned = ((v - rem_off) // 128) * 128
    if rem_aligned:
        chunks.append((rem_off, rem_aligned))
    tail_off = rem_off + rem_aligned
    tail = v - tail_off

    @functools.partial(
        pl.kernel,
        mesh=mesh,
        out_type=[
            jax.ShapeDtypeStruct((m_rows, b), jnp.float32),
            jax.ShapeDtypeStruct((d * v,), jnp.float32),
        ],
        scratch_types=[
            pltpu.VMEM((b,), jnp.int32),
            pltpu.VMEM((b,), jnp.int32),
            pltpu.VMEM((b,), jnp.float32),
            pltpu.VMEM((_P1_CHUNK,), jnp.float32),
            pltpu.VMEM((1, max(tail, 1)), jnp.float32),
            pltpu.SemaphoreType.DMA,
            pltpu.SemaphoreType.DMA,
        ],
        compiler_params=pltpu.CompilerParams(needs_layout_passes=False),
    )
    def gather(table_hbm, idx_hbm, out_hbm, scr_hbm, idx0_v, idx1_v, vals_v,
               chunk_v, tail_v, s_idx, s_g):
        idx_b = [idx0_v, idx1_v]
        w = lax.axis_index("s") * _NUM_CORES + lax.axis_index("c")
        row = scr_hbm.at[pl.ds(w * v, v)]
        feat = table_hbm.at[w]

        ci = [None] * l_fields
        ci[0] = pltpu.async_copy(idx_hbm.at[0], idx_b[0], s_idx)

        for off, sz in chunks:
            pltpu.sync_copy(
                feat.at[pl.ds(off, sz)], chunk_v.at[pl.ds(0, sz)]
            )
            pltpu.sync_copy(
                chunk_v.at[pl.ds(0, sz)], row.at[pl.ds(off, sz)]
            )
        if tail:
            pltpu.sync_copy(
                table_hbm.at[pl.ds(w, 1), pl.ds(tail_off, tail)], tail_v
            )
            pltpu.sync_copy(tail_v.at[0], row.at[pl.ds(tail_off, tail)])

        for r in range(l_fields):
            ci[r].wait()
            cg = pltpu.async_copy(row.at[idx_b[r % 2]], vals_v, s_g)
            if r + 1 < l_fields:
                ci[r + 1] = pltpu.async_copy(
                    idx_hbm.at[r + 1], idx_b[(r + 1) % 2], s_idx
                )
            cg.wait()
            pltpu.sync_copy(vals_v, out_hbm.at[r * d + w])

    return gather(table_t, idx_t)[0]


def kernel(inputs, embedding):
    b, l = inputs.shape
    v, d = embedding.shape
    idx_t = inputs.T.astype(jnp.int32)
    table_t = embedding.T
    out_t = _gather_t(table_t, idx_t, v)
    return out_t.T
